# baseline (device time: 29732 ns/iter reference)
import jax
import jax.numpy as jnp
from jax import lax
from jax.experimental import pallas as pl
from jax.experimental.pallas import tpu as pltpu

N_SPLIT = 8
N_STAGES = 4


def kernel(Q, K, V):
    b, q_len, h, d = Q.shape
    k_len = K.shape[1]
    kk = k_len // N_SPLIT
    hd = h * d
    scale = d ** -0.5
    K2 = K.reshape(b, k_len, hd)
    V2 = V.reshape(b, k_len, hd)

    def body(q_ref, k_ref, v_ref, o_ref,
             kv, vv, obuf, sbuf, kv_sems, osend, orecv, ssend, srecv):
        my_x = lax.axis_index("x")
        my_y = lax.axis_index("y")
        my_z = lax.axis_index("z")
        partners = [
            (my_x, 1 - my_y, my_z),
            (1 - my_x, my_y, my_z),
            (my_x, my_y, jnp.bitwise_xor(my_z, 1)),
            (my_x, my_y, jnp.bitwise_xor(my_z, 2)),
        ]

        start = (my_x * (N_SPLIT // 2) + my_z) * kk
        ck = pltpu.make_async_copy(
            k_ref.at[:, pl.ds(start, kk), :], kv, kv_sems.at[0])
        cv = pltpu.make_async_copy(
            v_ref.at[:, pl.ds(start, kk), :], vv, kv_sems.at[1])
        ck.start()
        cv.start()

        barrier_sem = pltpu.get_barrier_semaphore()
        for prt in partners:
            pl.semaphore_signal(
                barrier_sem, inc=1, device_id=prt,
                device_id_type=pl.DeviceIdType.MESH,
            )

        hp = lax.broadcasted_iota(jnp.int32, (h, hd), 1) // d
        hh = lax.broadcasted_iota(jnp.int32, (h, hd), 0)
        mask = (hp == hh).astype(jnp.float32)

        ck.wait()
        cv.wait()
        for bi in range(b):
            qb = q_ref[bi, 0]
            qbd = (jnp.concatenate([qb] * h, axis=1) * mask).astype(jnp.bfloat16)
            kb = kv[bi].astype(jnp.bfloat16)
            s = jax.lax.dot_general(
                qbd, kb, (((1,), (1,)), ((), ())),
                preferred_element_type=jnp.float32) * scale
            m = jnp.max(s, axis=1, keepdims=True)
            p = jnp.exp(s - m)
            l = jnp.sum(p, axis=1, keepdims=True)
            vb = vv[bi].astype(jnp.bfloat16)
            r = jax.lax.dot_general(
                p.astype(jnp.bfloat16), vb, (((1,), (0,)), ((), ())),
                preferred_element_type=jnp.float32)
            rm = r * mask
            o = rm[:, 0:d]
            for hi in range(1, h):
                o = o + rm[:, hi * d:(hi + 1) * d]
            obuf[0, bi] = o
            sbuf[0, bi] = jnp.concatenate([m, l], axis=1)

        pl.semaphore_wait(barrier_sem, N_STAGES)

        for s, prt in enumerate(partners):
            rdma_o = pltpu.make_async_remote_copy(
                src_ref=obuf.at[0], dst_ref=obuf.at[1 + s],
                send_sem=osend.at[s], recv_sem=orecv.at[s],
                device_id=prt, device_id_type=pl.DeviceIdType.MESH,
            )
            rdma_s = pltpu.make_async_remote_copy(
                src_ref=sbuf.at[0], dst_ref=sbuf.at[1 + s],
                send_sem=ssend.at[s], recv_sem=srecv.at[s],
                device_id=prt, device_id_type=pl.DeviceIdType.MESH,
            )
            rdma_o.start()
            rdma_s.start()
            rdma_o.wait()
            rdma_s.wait()

            m_me = sbuf[0, :, :, 0]
            l_me = sbuf[0, :, :, 1]
            o_me = obuf[0]
            m_pe = sbuf[1 + s, :, :, 0]
            l_pe = sbuf[1 + s, :, :, 1]
            o_pe = obuf[1 + s]

            m_new = jnp.maximum(m_me, m_pe)
            a_me = jnp.exp(m_me - m_new)
            a_pe = jnp.exp(m_pe - m_new)
            l_new = l_me * a_me + l_pe * a_pe
            o_new = o_me * a_me[..., None] + o_pe * a_pe[..., None]
            obuf[0] = o_new
            sbuf[0, :, :, 0] = m_new
            sbuf[0, :, :, 1] = l_new

        o_ref[:, 0] = obuf[0] / sbuf[0, :, :, 1][..., None]

    return pl.pallas_call(
        body,
        out_shape=jax.ShapeDtypeStruct((b, q_len, h, d), jnp.float32),
        in_specs=[
            pl.BlockSpec(memory_space=pltpu.VMEM),
            pl.BlockSpec(memory_space=pl.ANY),
            pl.BlockSpec(memory_space=pl.ANY),
        ],
        out_specs=pl.BlockSpec(memory_space=pltpu.VMEM),
        scratch_shapes=[
            pltpu.VMEM((b, kk, hd), jnp.float32),
            pltpu.VMEM((b, kk, hd), jnp.float32),
            pltpu.VMEM((1 + N_STAGES, b, h, d), jnp.float32),
            pltpu.VMEM((1 + N_STAGES, b, h, 2), jnp.float32),
            pltpu.SemaphoreType.DMA((2,)),
            pltpu.SemaphoreType.DMA((N_STAGES,)),
            pltpu.SemaphoreType.DMA((N_STAGES,)),
            pltpu.SemaphoreType.DMA((N_STAGES,)),
            pltpu.SemaphoreType.DMA((N_STAGES,)),
        ],
        compiler_params=pltpu.CompilerParams(collective_id=0),
    )(Q, K2, V2)


# device time: 18737 ns/iter; 1.5868x vs baseline; 1.5868x over previous
import jax
import jax.numpy as jnp
from jax import lax
from jax.experimental import pallas as pl
from jax.experimental.pallas import tpu as pltpu


def kernel(Q, K, V):
    b, q_len, h, d = Q.shape
    k_len = K.shape[1]
    hd = h * d
    bh = b * h
    bkk = b * k_len
    scale = d ** -0.5
    K2 = K.reshape(b, k_len, hd)
    V2 = V.reshape(b, k_len, hd)

    def body(q_ref, k_ref, v_ref, o_ref, pb, send_sem, recv_sem):
        my_x = lax.axis_index("x")
        my_y = lax.axis_index("y")
        my_z = lax.axis_index("z")
        peer = (my_x, 1 - my_y, my_z)

        barrier_sem = pltpu.get_barrier_semaphore()
        pl.semaphore_signal(barrier_sem, inc=1, device_id=peer,
                            device_id_type=pl.DeviceIdType.MESH)

        k3 = k_ref[...].reshape(bkk, hd)
        v3 = v_ref[...].reshape(bkk, hd)
        q2 = q_ref[...].reshape(bh, d)
        qt = jnp.concatenate([q2] * h, axis=1)
        rowh = lax.broadcasted_iota(jnp.int32, (bh, hd), 0) % h
        colh = lax.broadcasted_iota(jnp.int32, (bh, hd), 1) // d
        hmask = (rowh == colh).astype(jnp.float32)
        qbd = (qt * hmask).astype(jnp.bfloat16)

        s = jax.lax.dot_general(
            qbd, k3.astype(jnp.bfloat16), (((1,), (1,)), ((), ())),
            preferred_element_type=jnp.float32) * scale
        rowb = lax.broadcasted_iota(jnp.int32, (bh, bkk), 0) // h
        colb = lax.broadcasted_iota(jnp.int32, (bh, bkk), 1) // k_len
        s = jnp.where(rowb == colb, s, -1e30)
        m = jnp.max(s, axis=1, keepdims=True)
        p = jnp.exp(s - m)
        l = jnp.sum(p, axis=1, keepdims=True)
        r = jax.lax.dot_general(
            p.astype(jnp.bfloat16), v3.astype(jnp.bfloat16),
            (((1,), (0,)), ((), ())),
            preferred_element_type=jnp.float32)
        rm = r * hmask
        o = rm[:, 0:d]
        for hi in range(1, h):
            o = o + rm[:, hi * d:(hi + 1) * d]
        pb[0, :, 0:d] = o
        pb[0, :, d:d + 1] = m
        pb[0, :, d + 1:d + 2] = l

        pl.semaphore_wait(barrier_sem, 1)
        rdma = pltpu.make_async_remote_copy(
            src_ref=pb.at[0], dst_ref=pb.at[1],
            send_sem=send_sem, recv_sem=recv_sem,
            device_id=peer, device_id_type=pl.DeviceIdType.MESH,
        )
        rdma.start()
        rdma.wait()

        m_me = pb[0, :, d:d + 1]
        l_me = pb[0, :, d + 1:d + 2]
        m_pe = pb[1, :, d:d + 1]
        l_pe = pb[1, :, d + 1:d + 2]
        m_new = jnp.maximum(m_me, m_pe)
        a_me = jnp.exp(m_me - m_new)
        a_pe = jnp.exp(m_pe - m_new)
        l_new = l_me * a_me + l_pe * a_pe
        on = (pb[0, :, 0:d] * a_me + pb[1, :, 0:d] * a_pe) / l_new
        for bi in range(b):
            o_ref[bi, 0] = on[bi * h:(bi + 1) * h, :]

    return pl.pallas_call(
        body,
        out_shape=jax.ShapeDtypeStruct((b, q_len, h, d), jnp.float32),
        in_specs=[
            pl.BlockSpec(memory_space=pltpu.VMEM),
            pl.BlockSpec(memory_space=pltpu.VMEM),
            pl.BlockSpec(memory_space=pltpu.VMEM),
        ],
        out_specs=pl.BlockSpec(memory_space=pltpu.VMEM),
        scratch_shapes=[
            pltpu.VMEM((2, bh, d + 2), jnp.float32),
            pltpu.SemaphoreType.DMA,
            pltpu.SemaphoreType.DMA,
        ],
        compiler_params=pltpu.CompilerParams(collective_id=0),
    )(Q, K2, V2)
